# baseline (speedup 1.0000x reference)
import jax
import jax.numpy as jnp
from jax import lax
from jax.experimental import pallas as pl
from jax.experimental.pallas import tpu as pltpu
from jax.experimental.pallas import tpu_sc as plsc

_K = 512
_NROWS = 128
_NCOLS = 32768
_L = 16
_NW = 32
_RPW = _NROWS // _NW
_CAP = 8192
_CAP2 = 1024
_MIN32 = -(2 ** 31)
_C0_U = 0x40000000 ^ _MIN32


def _monokey(bits):
    return bits ^ (lax.shift_right_arithmetic(bits, 31) & jnp.int32(0x7FFFFFFF))


def _sc_body(x_hbm, out_hbm, row0, row1, outb, pos0, pos1, cand2_v, hist_v,
             sem_in, sem_out):
    wid = lax.axis_index("s") * 2 + lax.axis_index("c")
    base = wid * _RPW
    zeros16 = jnp.zeros((_L,), jnp.int32)
    ones16 = jnp.ones((_L,), jnp.int32)
    zerosf = jnp.zeros((_L,), jnp.float32)
    lanes = lax.iota(jnp.int32, _L)
    pad16 = jnp.full((_L,), _NCOLS, jnp.int32)
    rows = (row0, row1)
    poss = (pos0, pos1)

    row0[pl.ds(_NCOLS, _L)] = zerosf
    row1[pl.ds(_NCOLS, _L)] = zerosf

    in_desc = [None] * _RPW
    in_desc[0] = pltpu.async_copy(x_hbm.at[base], row0.at[pl.ds(0, _NCOLS)],
                                  sem_in)
    out_desc = None
    prev_ok = jnp.bool_(False)
    prev_nv = jnp.int32(0)

    for r in range(_RPW):
        row_v = rows[r & 1]
        pos_v = poss[r & 1]
        in_desc[r].wait()
        if r + 1 < _RPW:
            in_desc[r + 1] = pltpu.async_copy(
                x_hbm.at[base + (r + 1)],
                rows[(r + 1) & 1].at[pl.ds(0, _NCOLS)], sem_in)

        @plsc.parallel_loop(0, _NCOLS, step=_L, unroll=16, carry=zeros16)
        def off_vec(i, ov):
            v = row_v[pl.ds(i, _L)]
            m = v >= jnp.float32(2.0)
            mi = m.astype(jnp.int32)
            pos = ov + plsc.cumsum(mi) - mi
            plsc.store_scatter(pos_v, [pos], i + lanes,
                               mask=m & (pos < jnp.int32(_CAP)))
            return ov + plsc.all_reduce_population_count(m)

        cnt0 = off_vec[0]
        ok = (cnt0 >= _K) & (cnt0 <= _CAP)
        pos_v[pl.ds(jnp.minimum(cnt0, jnp.int32(_CAP)), _L)] = pad16
        nv2 = (jnp.minimum(cnt0, jnp.int32(_CAP)) + (_L - 1)) // _L

        @plsc.parallel_loop(0, 17 * _L, step=_L)
        def _hclr(i):
            hist_v[pl.ds(i, _L)] = jnp.zeros((_L,), jnp.int32)

        @plsc.parallel_loop(0, nv2 * _L, step=_L)
        def _hpass(j):
            v = plsc.load_gather(row_v, [pos_v[pl.ds(j, _L)]])
            bits = plsc.bitcast(v, jnp.int32)
            b = lax.shift_right_arithmetic(bits - jnp.int32(0x40000000), 19)
            b = jnp.clip(b, 0, 16)
            plsc.addupdate_scatter(hist_v, [b * _L + lanes], ones16)

        def hscan(i, carry):
            acc, b_star, above, h_star = carry
            b = jnp.int32(16) - i
            s = jnp.sum(hist_v[pl.ds(b * _L, _L)])
            acc2 = acc + s
            hit = (acc < _K) & (acc2 >= _K)
            return (acc2,
                    jnp.where(hit, b, b_star),
                    jnp.where(hit, acc, above),
                    jnp.where(hit, s, h_star))

        _, b_star, above, h_star = lax.fori_loop(
            0, 17, hscan, (jnp.int32(0),) * 4)
        histable = ok & (b_star < 16) & (h_star <= jnp.int32(_CAP2))

        def hist_thr():
            bkey = jnp.int32(0x40000000) + (b_star << 19)

            @plsc.parallel_loop(0, nv2 * _L, step=_L, carry=jnp.int32(0))
            def cnt2(j, off):
                p = pos_v[pl.ds(j, _L)]
                bits = plsc.bitcast(plsc.load_gather(row_v, [p]), jnp.int32)
                m = (bits >= bkey) & (bits < bkey + jnp.int32(1 << 19))
                plsc.store_compressed(
                    cand2_v.at[pl.ds(off, _L)], p,
                    mask=m & (off < jnp.int32(_CAP2 - _L + 1)))
                return off + plsc.all_reduce_population_count(m)[0]

            cand2_v[pl.ds(jnp.minimum(cnt2, jnp.int32(_CAP2)), _L)] = pad16
            nv3 = (cnt2 + (_L - 1)) // _L
            r2 = _K - above

            def bis(i, t_u):
                cand_u = t_u | (jnp.int32(1) << (jnp.int32(31) - i))
                cand_f = plsc.bitcast(
                    jnp.full((_L,), cand_u ^ jnp.int32(_MIN32), jnp.int32),
                    jnp.float32)

                @plsc.parallel_loop(0, nv3 * _L, step=_L, carry=zeros16)
                def cvec(j, a):
                    v = plsc.load_gather(row_v, [cand2_v[pl.ds(j, _L)]])
                    return a + jnp.where(v >= cand_f, 1, 0)

                cnt = jnp.sum(cvec)
                return jnp.where(cnt >= r2, cand_u, t_u)

            return lax.fori_loop(13, 32, bis, bkey ^ jnp.int32(_MIN32))

        def fast_thr():
            def bis(i, t_u):
                cand_u = t_u | (jnp.int32(1) << (jnp.int32(31) - i))
                cand_f = plsc.bitcast(
                    jnp.full((_L,), cand_u ^ jnp.int32(_MIN32), jnp.int32),
                    jnp.float32)

                @plsc.parallel_loop(0, nv2 * _L, step=_L, carry=zeros16)
                def cvec(j, a):
                    v = plsc.load_gather(row_v, [pos_v[pl.ds(j, _L)]])
                    return a + jnp.where(v >= cand_f, 1, 0)

                cnt = jnp.sum(cvec)
                return jnp.where(cnt >= _K, cand_u, t_u)

            return lax.fori_loop(2, 32, bis, jnp.int32(_C0_U))

        def slow_thr():
            def bis(i, t_u):
                cand_u = t_u | (jnp.int32(1) << (jnp.int32(31) - i))
                cand = cand_u ^ jnp.int32(_MIN32)

                @plsc.parallel_loop(0, _NCOLS, step=_L, unroll=4, carry=zeros16)
                def cvec(j, a):
                    key = _monokey(
                        plsc.bitcast(row_v[pl.ds(j, _L)], jnp.int32))
                    return a + jnp.where(key >= cand, 1, 0)

                cnt = jnp.sum(cvec)
                return jnp.where(cnt >= _K, cand_u, t_u)

            return lax.fori_loop(0, 32, bis, jnp.int32(0))

        t_u = lax.cond(
            histable, hist_thr, lambda: lax.cond(ok, fast_thr, slow_thr))
        thr = jnp.maximum(t_u ^ jnp.int32(_MIN32), jnp.int32(1))
        thr_f = plsc.bitcast(jnp.full((_L,), thr, jnp.int32), jnp.float32)

        if out_desc is not None:
            out_desc.wait()

        if r == 0:
            @plsc.parallel_loop(0, _NCOLS, step=_L, unroll=8)
            def _z0(j):
                outb[pl.ds(j, _L)] = zerosf
        else:
            prev_pos = poss[(r - 1) & 1]
            p_nv = prev_nv

            def rezero_scatter():
                @plsc.parallel_loop(0, p_nv * _L, step=_L)
                def _zs(j):
                    plsc.store_scatter(outb, [prev_pos[pl.ds(j, _L)]], zerosf)
                return 0

            def rezero_full():
                @plsc.parallel_loop(0, _NCOLS, step=_L, unroll=8)
                def _zf(j):
                    outb[pl.ds(j, _L)] = zerosf
                return 0

            lax.cond(prev_ok, rezero_scatter, rezero_full)

        def write_scatter():
            @plsc.parallel_loop(0, nv2 * _L, step=_L)
            def _ws(j):
                p = pos_v[pl.ds(j, _L)]
                v = plsc.load_gather(row_v, [p])
                plsc.store_scatter(outb, [p], v, mask=v >= thr_f)
            return 0

        def write_full():
            @plsc.parallel_loop(0, _NCOLS, step=_L, unroll=8)
            def _wf(j):
                v = row_v[pl.ds(j, _L)]
                outb[pl.ds(j, _L)] = jnp.where(v >= thr_f, v, 0.0)
            return 0

        lax.cond(ok, write_scatter, write_full)
        out_desc = pltpu.async_copy(outb.at[pl.ds(0, _NCOLS)],
                                    out_hbm.at[base + r], sem_out)
        prev_ok = ok
        prev_nv = nv2

    out_desc.wait()


def kernel(x):
    mesh = plsc.VectorSubcoreMesh(
        core_axis_name="c", subcore_axis_name="s", num_cores=2, num_subcores=16)
    f = pl.kernel(
        _sc_body,
        out_type=jax.ShapeDtypeStruct((_NROWS, _NCOLS), jnp.float32),
        mesh=mesh,
        compiler_params=pltpu.CompilerParams(needs_layout_passes=False),
        scratch_types=[
            pltpu.VMEM((_NCOLS + _L,), jnp.float32),
            pltpu.VMEM((_NCOLS + _L,), jnp.float32),
            pltpu.VMEM((_NCOLS + _L,), jnp.float32),
            pltpu.VMEM((_CAP + _L,), jnp.int32),
            pltpu.VMEM((_CAP + _L,), jnp.int32),
            pltpu.VMEM((_CAP2 + _L,), jnp.int32),
            pltpu.VMEM((17 * _L,), jnp.int32),
            pltpu.SemaphoreType.DMA,
            pltpu.SemaphoreType.DMA,
        ],
    )
    return f(x)


# speedup vs baseline: 1.1105x; 1.1105x over previous
import jax
import jax.numpy as jnp
from jax import lax
from jax.experimental import pallas as pl
from jax.experimental.pallas import tpu as pltpu
from jax.experimental.pallas import tpu_sc as plsc

_K = 512
_NROWS = 128
_NCOLS = 32768
_L = 16
_NW = 32
_RPW = _NROWS // _NW
_CAP = 8192
_CAP2 = 1024
_MIN32 = -(2 ** 31)
_C0_U = 0x40000000 ^ _MIN32


def _monokey(bits):
    return bits ^ (lax.shift_right_arithmetic(bits, 31) & jnp.int32(0x7FFFFFFF))


def _sc_body(x_hbm, out_hbm, row0, row1, outb, pos0, pos1, cand2_v, hist_v,
             sem_in, sem_out):
    wid = lax.axis_index("s") * 2 + lax.axis_index("c")
    base = wid * _RPW
    zeros16 = jnp.zeros((_L,), jnp.int32)
    ones16 = jnp.ones((_L,), jnp.int32)
    zerosf = jnp.zeros((_L,), jnp.float32)
    lanes = lax.iota(jnp.int32, _L)
    pad16 = jnp.full((_L,), _NCOLS, jnp.int32)
    rows = (row0, row1)
    poss = (pos0, pos1)

    row0[pl.ds(_NCOLS, _L)] = zerosf
    row1[pl.ds(_NCOLS, _L)] = zerosf

    in_desc = [None] * _RPW
    in_desc[0] = pltpu.async_copy(x_hbm.at[base], row0.at[pl.ds(0, _NCOLS)],
                                  sem_in)
    out_desc = None
    prev_ok = jnp.bool_(False)
    prev_nv = jnp.int32(0)

    for r in range(_RPW):
        row_v = rows[r & 1]
        pos_v = poss[r & 1]
        in_desc[r].wait()
        if r + 1 < _RPW:
            in_desc[r + 1] = pltpu.async_copy(
                x_hbm.at[base + (r + 1)],
                rows[(r + 1) & 1].at[pl.ds(0, _NCOLS)], sem_in)

        @plsc.parallel_loop(0, _NCOLS, step=_L, unroll=8, carry=zeros16)
        def off_vec(i, ov):
            v = row_v[pl.ds(i, _L)]
            m = v >= jnp.float32(2.0)
            mi = m.astype(jnp.int32)
            pos = ov + plsc.cumsum(mi) - mi
            plsc.store_scatter(pos_v, [pos], i + lanes,
                               mask=m & (pos < jnp.int32(_CAP)))
            return ov + plsc.all_reduce_population_count(m)

        cnt0 = off_vec[0]
        ok = (cnt0 >= _K) & (cnt0 <= _CAP)
        pos_v[pl.ds(jnp.minimum(cnt0, jnp.int32(_CAP)), _L)] = pad16
        nv2 = (jnp.minimum(cnt0, jnp.int32(_CAP)) + (_L - 1)) // _L

        @plsc.parallel_loop(0, 17 * _L, step=_L)
        def _hclr(i):
            hist_v[pl.ds(i, _L)] = jnp.zeros((_L,), jnp.int32)

        @plsc.parallel_loop(0, nv2 * _L, step=_L)
        def _hpass(j):
            v = plsc.load_gather(row_v, [pos_v[pl.ds(j, _L)]])
            bits = plsc.bitcast(v, jnp.int32)
            b = lax.shift_right_arithmetic(bits - jnp.int32(0x40000000), 19)
            b = jnp.clip(b, 0, 16)
            plsc.addupdate_scatter(hist_v, [b * _L + lanes], ones16)

        def hscan(i, carry):
            acc, b_star, above, h_star = carry
            b = jnp.int32(16) - i
            s = jnp.sum(hist_v[pl.ds(b * _L, _L)])
            acc2 = acc + s
            hit = (acc < _K) & (acc2 >= _K)
            return (acc2,
                    jnp.where(hit, b, b_star),
                    jnp.where(hit, acc, above),
                    jnp.where(hit, s, h_star))

        _, b_star, above, h_star = lax.fori_loop(
            0, 17, hscan, (jnp.int32(0),) * 4)
        histable = ok & (b_star < 16) & (h_star <= jnp.int32(_CAP2))

        def hist_thr():
            bkey = jnp.int32(0x40000000) + (b_star << 19)

            @plsc.parallel_loop(0, nv2 * _L, step=_L, carry=zeros16)
            def off2_vec(j, ov):
                p = pos_v[pl.ds(j, _L)]
                bits = plsc.bitcast(plsc.load_gather(row_v, [p]), jnp.int32)
                m = (bits >= bkey) & (bits < bkey + jnp.int32(1 << 19))
                mi = m.astype(jnp.int32)
                pos = ov + plsc.cumsum(mi) - mi
                plsc.store_scatter(cand2_v, [pos], p,
                                   mask=m & (pos < jnp.int32(_CAP2)))
                return ov + plsc.all_reduce_population_count(m)

            cnt2 = off2_vec[0]
            cand2_v[pl.ds(jnp.minimum(cnt2, jnp.int32(_CAP2)), _L)] = pad16
            nv3 = (cnt2 + (_L - 1)) // _L
            r2 = _K - above

            def bis(i, t_u):
                cand_u = t_u | (jnp.int32(1) << (jnp.int32(31) - i))
                cand_f = plsc.bitcast(cand_u ^ jnp.int32(_MIN32), jnp.float32)

                @plsc.parallel_loop(0, nv3 * _L, step=_L, carry=zeros16)
                def cvec(j, a):
                    v = plsc.load_gather(row_v, [cand2_v[pl.ds(j, _L)]])
                    return a + plsc.all_reduce_population_count(v >= cand_f)

                return jnp.where(cvec >= r2, cand_u, t_u)

            t_u = lax.fori_loop(
                13, 32, bis,
                jnp.full((_L,), bkey ^ jnp.int32(_MIN32), jnp.int32))
            return t_u[0]

        def fast_thr():
            def bis(i, t_u):
                cand_u = t_u | (jnp.int32(1) << (jnp.int32(31) - i))
                cand_f = plsc.bitcast(cand_u ^ jnp.int32(_MIN32), jnp.float32)

                @plsc.parallel_loop(0, nv2 * _L, step=_L, carry=zeros16)
                def cvec(j, a):
                    v = plsc.load_gather(row_v, [pos_v[pl.ds(j, _L)]])
                    return a + plsc.all_reduce_population_count(v >= cand_f)

                return jnp.where(cvec >= _K, cand_u, t_u)

            t_u = lax.fori_loop(
                2, 32, bis, jnp.full((_L,), _C0_U, jnp.int32))
            return t_u[0]

        def slow_thr():
            def bis(i, t_u):
                cand_u = t_u | (jnp.int32(1) << (jnp.int32(31) - i))
                cand = cand_u ^ jnp.int32(_MIN32)

                @plsc.parallel_loop(0, _NCOLS, step=_L, unroll=4, carry=zeros16)
                def cvec(j, a):
                    key = _monokey(
                        plsc.bitcast(row_v[pl.ds(j, _L)], jnp.int32))
                    return a + jnp.where(key >= cand, 1, 0)

                cnt = jnp.sum(cvec)
                return jnp.where(cnt >= _K, cand_u, t_u)

            return lax.fori_loop(0, 32, bis, jnp.int32(0))

        t_u = lax.cond(
            histable, hist_thr, lambda: lax.cond(ok, fast_thr, slow_thr))
        thr = jnp.maximum(t_u ^ jnp.int32(_MIN32), jnp.int32(1))
        thr_f = plsc.bitcast(jnp.full((_L,), thr, jnp.int32), jnp.float32)

        if out_desc is not None:
            out_desc.wait()

        if r == 0:
            @plsc.parallel_loop(0, _NCOLS, step=_L, unroll=8)
            def _z0(j):
                outb[pl.ds(j, _L)] = zerosf
        else:
            prev_pos = poss[(r - 1) & 1]
            p_nv = prev_nv

            def rezero_scatter():
                @plsc.parallel_loop(0, p_nv * _L, step=_L)
                def _zs(j):
                    plsc.store_scatter(outb, [prev_pos[pl.ds(j, _L)]], zerosf)
                return 0

            def rezero_full():
                @plsc.parallel_loop(0, _NCOLS, step=_L, unroll=8)
                def _zf(j):
                    outb[pl.ds(j, _L)] = zerosf
                return 0

            lax.cond(prev_ok, rezero_scatter, rezero_full)

        def write_scatter():
            @plsc.parallel_loop(0, nv2 * _L, step=_L)
            def _ws(j):
                p = pos_v[pl.ds(j, _L)]
                v = plsc.load_gather(row_v, [p])
                plsc.store_scatter(outb, [p], v, mask=v >= thr_f)
            return 0

        def write_full():
            @plsc.parallel_loop(0, _NCOLS, step=_L, unroll=8)
            def _wf(j):
                v = row_v[pl.ds(j, _L)]
                outb[pl.ds(j, _L)] = jnp.where(v >= thr_f, v, 0.0)
            return 0

        lax.cond(ok, write_scatter, write_full)
        out_desc = pltpu.async_copy(outb.at[pl.ds(0, _NCOLS)],
                                    out_hbm.at[base + r], sem_out)
        prev_ok = ok
        prev_nv = nv2

    out_desc.wait()


def kernel(x):
    mesh = plsc.VectorSubcoreMesh(
        core_axis_name="c", subcore_axis_name="s", num_cores=2, num_subcores=16)
    f = pl.kernel(
        _sc_body,
        out_type=jax.ShapeDtypeStruct((_NROWS, _NCOLS), jnp.float32),
        mesh=mesh,
        compiler_params=pltpu.CompilerParams(needs_layout_passes=False),
        scratch_types=[
            pltpu.VMEM((_NCOLS + _L,), jnp.float32),
            pltpu.VMEM((_NCOLS + _L,), jnp.float32),
            pltpu.VMEM((_NCOLS + _L,), jnp.float32),
            pltpu.VMEM((_CAP + _L,), jnp.int32),
            pltpu.VMEM((_CAP + _L,), jnp.int32),
            pltpu.VMEM((_CAP2 + _L,), jnp.int32),
            pltpu.VMEM((17 * _L,), jnp.int32),
            pltpu.SemaphoreType.DMA,
            pltpu.SemaphoreType.DMA,
        ],
    )
    return f(x)
